# Initial kernel scaffold; baseline (speedup 1.0000x reference)
import jax
import jax.numpy as jnp
from jax.experimental import pallas as pl


def kernel(x, edge_index, batch, emb, vec_random, mlp_params, bn_params):
    raise NotImplementedError("write your pallas kernel here")


# speedup vs baseline: 2.3771x; 2.3771x over previous
import functools

import jax
import jax.numpy as jnp
from jax import lax
from jax.experimental import pallas as pl
from jax.experimental.pallas import tpu as pltpu
from jax.experimental.pallas import tpu_sc as plsc

N = 10000
E = 160000
NUM_EMB = 11868
IN_DIM = 300
HID = 128
NUM_LAYERS = 3
NUM_GRAPHS = 128

NC = 2
NS = 16
NP = 10240
VCAP = NUM_EMB + 2
D1 = 320
S1 = 4
DQ1 = D1 // S1
S2 = 2
DQ2 = HID // S2
EP = 163840
ECHUNK = 128
CHUNKS_PER_TILE = EP // (NS * ECHUNK)
ROWS_PER_TILE = NP // NS
NODE_CHUNKS = ROWS_PER_TILE // ECHUNK

_SC_PARAMS = pltpu.CompilerParams(use_tc_tiling_on_sc=False)


def _emb_body(nq, vh_hbm, idx_hbm, out_hbm, idx_v, rows_v, gsem):
    h = lax.axis_index("c")
    s = lax.axis_index("s")
    for ql in range(nq):
        q = h * nq + ql
        pltpu.sync_copy(idx_hbm.at[q * NS + s], idx_v)
        out_base = q * NP + s * ROWS_PER_TILE
        cur = pltpu.async_copy(vh_hbm.at[idx_v.at[0]], rows_v.at[0], gsem)
        for j in range(NODE_CHUNKS):
            cur.wait()
            if j + 1 < NODE_CHUNKS:
                nxt = pltpu.async_copy(
                    vh_hbm.at[idx_v.at[j + 1]], rows_v.at[(j + 1) % 2], gsem)
            pltpu.sync_copy(rows_v.at[j % 2],
                            out_hbm.at[pl.ds(out_base + j * ECHUNK, ECHUNK)])
            if j + 1 < NODE_CHUNKS:
                cur = nxt


@functools.partial(jax.jit, static_argnums=(2, 3))
def _emb_gather(vh, idx, nslab, dq):
    mesh = plsc.VectorSubcoreMesh(core_axis_name="c", subcore_axis_name="s")
    f = pl.kernel(
        functools.partial(_emb_body, nslab // NC),
        out_type=jax.ShapeDtypeStruct((nslab * NP, dq), jnp.float32),
        mesh=mesh,
        scratch_types=[
            pltpu.VMEM((NODE_CHUNKS, ECHUNK), jnp.int32),
            pltpu.VMEM((2, ECHUNK, dq), jnp.float32),
            pltpu.SemaphoreType.DMA,
        ],
        compiler_params=_SC_PARAMS,
    )
    return f(vh, idx)


def _agg_body(nq, z_hbm, src_hbm, dst_hbm, zer_hbm, out_hbm,
              src_v, dst_v, rows_v, acc, gsem):
    h = lax.axis_index("c")
    s = lax.axis_index("s")
    pltpu.sync_copy(dst_hbm.at[pl.ds(s * CHUNKS_PER_TILE, CHUNKS_PER_TILE)],
                    dst_v)
    for ql in range(nq):
        q = h * nq + ql
        pltpu.sync_copy(
            src_hbm.at[pl.ds((q * NS + s) * CHUNKS_PER_TILE,
                             CHUNKS_PER_TILE)],
            src_v)
        pltpu.sync_copy(zer_hbm,
                        acc.at[pl.ds(s * ROWS_PER_TILE, ROWS_PER_TILE)])
        plsc.subcore_barrier()

        def step(j, carry):
            pltpu.async_copy(z_hbm.at[src_v.at[j]], rows_v.at[0], gsem).wait()
            pltpu.sync_copy(rows_v.at[0], acc.at[dst_v.at[j]], add=True)
            return carry

        lax.fori_loop(0, CHUNKS_PER_TILE, step, 0)
        plsc.subcore_barrier()
        pltpu.sync_copy(
            acc.at[pl.ds(s * ROWS_PER_TILE, ROWS_PER_TILE)],
            out_hbm.at[pl.ds(q * NP + s * ROWS_PER_TILE, ROWS_PER_TILE)])


@functools.partial(jax.jit, static_argnums=(4, 5))
def _edge_agg(z2, srcs, dstw, zer, nslab, dq):
    mesh = plsc.VectorSubcoreMesh(core_axis_name="c", subcore_axis_name="s")
    f = pl.kernel(
        functools.partial(_agg_body, nslab // NC),
        out_type=jax.ShapeDtypeStruct((nslab * NP, dq), jnp.float32),
        mesh=mesh,
        scratch_types=[
            pltpu.VMEM((CHUNKS_PER_TILE, ECHUNK), jnp.int32),
            pltpu.VMEM((CHUNKS_PER_TILE, ECHUNK), jnp.int32),
            pltpu.VMEM((2, ECHUNK, dq), jnp.float32),
            pltpu.VMEM_SHARED((NP, dq), jnp.float32),
            pltpu.SemaphoreType.DMA,
        ],
        compiler_params=_SC_PARAMS,
    )
    return f(z2, srcs, dstw, zer)


def _dense_body(nslab, dq, z_ref, a_ref, w1_ref, b1_ref, w2_ref, b2_ref,
                gam_ref, bet_ref, bt_ref, zout_ref, g_ref):
    z = z_ref[...]
    a = a_ref[...]
    w1 = w1_ref[...]
    t = b1_ref[...]
    for q in range(nslab):
        hq = z[q * NP:(q + 1) * NP] + a[q * NP:(q + 1) * NP]
        t = t + jnp.dot(hq, w1[q * dq:(q + 1) * dq],
                        preferred_element_type=jnp.float32)
    t = jnp.maximum(t, 0.0)
    t = jnp.dot(t, w2_ref[...], preferred_element_type=jnp.float32) + b2_ref[...]
    t = jnp.maximum(t, 0.0)
    mask = lax.broadcasted_iota(jnp.int32, (NP, 1), 0) < N
    tm = jnp.where(mask, t, 0.0)
    mu = jnp.sum(tm, axis=0, keepdims=True) * (1.0 / N)
    d = jnp.where(mask, t - mu, 0.0)
    var = jnp.sum(d * d, axis=0, keepdims=True) * (1.0 / N)
    zz = gam_ref[...] * d * lax.rsqrt(var + 1e-5) + bet_ref[...]
    zz = jnp.where(mask, zz, 0.0)
    zout_ref[pl.ds(0, NP)] = zz[:, :DQ2]
    zout_ref[pl.ds(NP, NP)] = zz[:, DQ2:]
    bt = bt_ref[...]
    pt = (lax.broadcasted_iota(jnp.int32, (NUM_GRAPHS, NP), 0)
          == bt).astype(jnp.float32)
    pooled = lax.dot_general(pt, zz, (((1,), (0,)), ((), ())),
                             preferred_element_type=jnp.float32)
    cnt = jnp.sum(pt, axis=1, keepdims=True)
    g_ref[...] = pooled / jnp.maximum(cnt, 1.0)


@functools.partial(jax.jit, static_argnums=(8, 9))
def _dense(z2, a2, w1, b1, w2, b2, gam, bet, nslab, dq, bt):
    f = pl.pallas_call(
        functools.partial(_dense_body, nslab, dq),
        out_shape=(
            jax.ShapeDtypeStruct((2 * NP, DQ2), jnp.float32),
            jax.ShapeDtypeStruct((NUM_GRAPHS, HID), jnp.float32),
        ),
    )
    return f(z2, a2, w1, b1, w2, b2, gam, bet, bt)


def kernel(x, edge_index, batch, emb, vec_random, mlp_params, bn_params):
    f32 = jnp.float32
    vec_all = jnp.concatenate([emb, vec_random,
                               jnp.zeros((1, IN_DIM), f32)], axis=0)
    vec_all = jnp.pad(vec_all, ((0, 0), (0, D1 - IN_DIM)))
    vh = jnp.concatenate(
        [vec_all[:, q * DQ1:(q + 1) * DQ1] for q in range(S1)], axis=0)

    x_pad = jnp.concatenate(
        [x[:, 0], jnp.full((NP - N,), NUM_EMB + 1, jnp.int32)])
    emb_idx = jnp.concatenate(
        [x_pad + q * VCAP for q in range(S1)]).reshape(
            S1 * NS, NODE_CHUNKS, ECHUNK)

    src_pad = jnp.concatenate(
        [edge_index[0], jnp.full((EP - E,), N, jnp.int32)])
    dst_pad = jnp.concatenate(
        [edge_index[1], jnp.full((EP - E,), N, jnp.int32)])
    srcw = src_pad.reshape(-1, ECHUNK)
    src_s1 = jnp.concatenate([srcw + q * NP for q in range(S1)], axis=0)
    src_s2 = jnp.concatenate([srcw + q * NP for q in range(S2)], axis=0)
    dstw = dst_pad.reshape(-1, ECHUNK)

    zer1 = jnp.zeros((ROWS_PER_TILE, DQ1), f32)
    zer2 = jnp.zeros((ROWS_PER_TILE, DQ2), f32)
    bt = jnp.concatenate(
        [batch, jnp.full((NP - N,), NUM_GRAPHS, jnp.int32)]).reshape(1, NP)

    z2 = _emb_gather(vh, emb_idx, S1, DQ1)

    zs = []
    gs = []
    for i in range(NUM_LAYERS):
        (W1, b1, W2, b2), (gamma, beta) = mlp_params[i], bn_params[i]
        if i == 0:
            W1 = jnp.pad(W1, ((0, D1 - IN_DIM), (0, 0)))
            nslab, dq, srcs, zer = S1, DQ1, src_s1, zer1
        else:
            nslab, dq, srcs, zer = S2, DQ2, src_s2, zer2
        a2 = _edge_agg(z2, srcs, dstw, zer, nslab, dq)
        z2, g = _dense(z2, a2, W1, b1.reshape(1, HID), W2,
                       b2.reshape(1, HID), gamma.reshape(1, HID),
                       beta.reshape(1, HID), nslab, dq, bt)
        zs.append(jnp.concatenate([z2[:N], z2[NP:NP + N]], axis=1))
        gs.append(g)
    return (jnp.concatenate(zs, axis=1), jnp.concatenate(gs, axis=1))


# speedup vs baseline: 2.6214x; 1.1028x over previous
import functools

import jax
import jax.numpy as jnp
from jax import lax
from jax.experimental import pallas as pl
from jax.experimental.pallas import tpu as pltpu
from jax.experimental.pallas import tpu_sc as plsc

N = 10000
E = 160000
NUM_EMB = 11868
IN_DIM = 300
HID = 128
NUM_LAYERS = 3
NUM_GRAPHS = 128

NC = 2
NS = 16
NP = 10240
VCAP = NUM_EMB + 2
D1 = 320
DQ = 64
S1 = D1 // DQ
NQ1 = 3
S2 = HID // DQ
EP = 163840
ECHUNK = 128
EROWS = EP // ECHUNK
ROWS_PER_TILE = NP // NS
NODE_CHUNKS = ROWS_PER_TILE // ECHUNK

_SC_PARAMS = pltpu.CompilerParams(use_tc_tiling_on_sc=False)
_ZV = 16


def _zero_rows(buf, nrow, dq):
    zv = jnp.zeros((_ZV,), jnp.float32)

    def row(i, c):
        for k in range(dq // _ZV):
            buf[i, pl.ds(k * _ZV, _ZV)] = zv
        return c

    lax.fori_loop(0, nrow, row, 0)


def _emb_body(vh_hbm, idx_hbm, out_hbm, idx_v, rows_v, gsem):
    h = lax.axis_index("c")
    s = lax.axis_index("s")
    for ql in range(NQ1):
        q = h * NQ1 + ql

        @pl.when(q < S1)
        def _():
            pltpu.sync_copy(idx_hbm.at[q * NS + s], idx_v)
            out_base = q * NP + s * ROWS_PER_TILE
            cur = pltpu.async_copy(vh_hbm.at[idx_v.at[0]], rows_v.at[0], gsem)
            for j in range(NODE_CHUNKS):
                cur.wait()
                if j + 1 < NODE_CHUNKS:
                    nxt = pltpu.async_copy(
                        vh_hbm.at[idx_v.at[j + 1]], rows_v.at[(j + 1) % 2],
                        gsem)
                pltpu.sync_copy(
                    rows_v.at[j % 2],
                    out_hbm.at[pl.ds(out_base + j * ECHUNK, ECHUNK)])
                if j + 1 < NODE_CHUNKS:
                    cur = nxt


@jax.jit
def _emb_gather(vh, idx):
    mesh = plsc.VectorSubcoreMesh(core_axis_name="c", subcore_axis_name="s")
    f = pl.kernel(
        _emb_body,
        out_type=jax.ShapeDtypeStruct((S1 * NP, DQ), jnp.float32),
        mesh=mesh,
        scratch_types=[
            pltpu.VMEM((NODE_CHUNKS, ECHUNK), jnp.int32),
            pltpu.VMEM((2, ECHUNK, DQ), jnp.float32),
            pltpu.SemaphoreType.DMA,
        ],
        compiler_params=_SC_PARAMS,
    )
    return f(vh, idx)


def _agg_body(nq, ns, dq, nbuf, z_hbm, src_hbm, dst_hbm, out_hbm,
              src_v, dst_v, rows_v, acc, gsem, ssem):
    h = lax.axis_index("c")
    s = lax.axis_index("s")
    nchunk = EROWS // NS
    base = s * nchunk
    pltpu.sync_copy(src_hbm.at[pl.ds(base, nchunk)], src_v)
    pltpu.sync_copy(dst_hbm.at[pl.ds(base, nchunk)], dst_v)

    def add_src(off):
        def adj(i, c):
            for k in range(ECHUNK // _ZV):
                sl = pl.ds(k * _ZV, _ZV)
                src_v[i, sl] = src_v[i, sl] + off
            return c

        lax.fori_loop(0, nchunk, adj, 0)

    def one_pass():
        def group(g, c):
            gds = []
            for b in range(nbuf):
                j = g * nbuf + b
                gds.append(pltpu.async_copy(
                    z_hbm.at[src_v.at[j]], rows_v.at[b], gsem))
            for d in gds:
                d.wait()
            sds = []
            for b in range(nbuf):
                j = g * nbuf + b
                sds.append(pltpu.async_copy(
                    rows_v.at[b], acc.at[dst_v.at[j]], ssem, add=True))
            for d in sds:
                d.wait()
            return c

        lax.fori_loop(0, nchunk // nbuf, group, 0)

    def zero_acc():
        _zero_rows(rows_v.at[0], ECHUNK, dq)
        for k in range(NODE_CHUNKS):
            pltpu.sync_copy(
                rows_v.at[0],
                acc.at[pl.ds(s * ROWS_PER_TILE + k * ECHUNK, ECHUNK)])

    def dump_acc(q):
        pltpu.sync_copy(
            acc.at[pl.ds(s * ROWS_PER_TILE, ROWS_PER_TILE)],
            out_hbm.at[pl.ds(q * NP + s * ROWS_PER_TILE, ROWS_PER_TILE)])

    for ql in range(nq):
        q = h * nq + ql

        @pl.when(q < ns)
        def _():
            add_src(q * NP if ql == 0 else NP)
            zero_acc()
            plsc.subcore_barrier()
            one_pass()
            plsc.subcore_barrier()
            dump_acc(q)
            if ql + 1 < nq:
                plsc.subcore_barrier()


@functools.partial(jax.jit, static_argnums=(3, 4, 5))
def _edge_agg(z2, srcw, dstw, nq, ns, nbuf):
    mesh = plsc.VectorSubcoreMesh(core_axis_name="c", subcore_axis_name="s")
    nchunk = EROWS // NS
    f = pl.kernel(
        functools.partial(_agg_body, nq, ns, DQ, nbuf),
        out_type=jax.ShapeDtypeStruct((ns * NP, DQ), jnp.float32),
        mesh=mesh,
        scratch_types=[
            pltpu.VMEM((nchunk, ECHUNK), jnp.int32),
            pltpu.VMEM((nchunk, ECHUNK), jnp.int32),
            pltpu.VMEM((nbuf, ECHUNK, DQ), jnp.float32),
            pltpu.VMEM_SHARED((NP, DQ), jnp.float32),
            pltpu.SemaphoreType.DMA,
            pltpu.SemaphoreType.DMA,
        ],
        compiler_params=_SC_PARAMS,
    )
    return f(z2, srcw, dstw)


def _dense_body(nslab, z_ref, a_ref, w1_ref, b1_ref, w2_ref, b2_ref,
                gam_ref, bet_ref, zout_ref, t_acc):
    q = pl.program_id(0)
    part = jnp.dot(z_ref[...] + a_ref[...], w1_ref[...],
                   preferred_element_type=jnp.float32)

    @pl.when(q == 0)
    def _():
        t_acc[...] = b1_ref[...] + part

    @pl.when(q > 0)
    def _():
        t_acc[...] = t_acc[...] + part

    @pl.when(q == nslab - 1)
    def _():
        t = jnp.maximum(t_acc[...], 0.0)
        t = (jnp.dot(t, w2_ref[...], preferred_element_type=jnp.float32)
             + b2_ref[...])
        t = jnp.maximum(t, 0.0)
        mask = lax.broadcasted_iota(jnp.int32, (NP, 1), 0) < N
        tm = jnp.where(mask, t, 0.0)
        mu = jnp.sum(tm, axis=0, keepdims=True) * (1.0 / N)
        d = jnp.where(mask, t - mu, 0.0)
        var = jnp.sum(d * d, axis=0, keepdims=True) * (1.0 / N)
        zz = gam_ref[...] * d * lax.rsqrt(var + 1e-5) + bet_ref[...]
        zz = jnp.where(mask, zz, 0.0)
        zout_ref[pl.ds(0, NP)] = zz[:, :DQ]
        zout_ref[pl.ds(NP, NP)] = zz[:, DQ:]


@functools.partial(jax.jit, static_argnums=(8,))
def _dense(z2, a2, w1, b1, w2, b2, gam, bet, first):
    nslab = S1 if first else S2
    full = lambda *shape: pl.BlockSpec(shape, lambda q: (0,) * len(shape))
    f = pl.pallas_call(
        functools.partial(_dense_body, nslab),
        grid=(nslab,),
        in_specs=[
            pl.BlockSpec((NP, DQ), lambda q: (q, 0)),
            pl.BlockSpec((NP, DQ), lambda q: (q, 0)),
            pl.BlockSpec((DQ, HID), lambda q: (q, 0)),
            full(1, HID), full(HID, HID), full(1, HID),
            full(1, HID), full(1, HID),
        ],
        out_specs=pl.BlockSpec((NC * NP, DQ), lambda q: (0, 0)),
        out_shape=jax.ShapeDtypeStruct((NC * NP, DQ), jnp.float32),
        scratch_shapes=[pltpu.VMEM((NP, HID), jnp.float32)],
    )
    return f(z2, a2, w1, b1, w2, b2, gam, bet)


def _pool_body(z1_ref, z2_ref, z3_ref, bt_ref, g_ref):
    bt = bt_ref[...]
    pt = (lax.broadcasted_iota(jnp.int32, (NUM_GRAPHS, NP), 0)
          == bt).astype(jnp.float32)
    cnt = jnp.sum(pt, axis=1, keepdims=True)
    inv = 1.0 / jnp.maximum(cnt, 1.0)
    for i, zr in enumerate([z1_ref, z2_ref, z3_ref]):
        z = zr[...]
        zz = jnp.concatenate([z[:NP], z[NP:]], axis=1)
        g_ref[:, pl.ds(i * HID, HID)] = inv * lax.dot_general(
            pt, zz, (((1,), (0,)), ((), ())),
            preferred_element_type=jnp.float32)


@jax.jit
def _pool(z1, z2, z3, bt):
    f = pl.pallas_call(
        _pool_body,
        out_shape=jax.ShapeDtypeStruct(
            (NUM_GRAPHS, NUM_LAYERS * HID), jnp.float32),
    )
    return f(z1, z2, z3, bt)


def kernel(x, edge_index, batch, emb, vec_random, mlp_params, bn_params):
    f32 = jnp.float32
    vec_all = jnp.concatenate([emb, vec_random,
                               jnp.zeros((1, IN_DIM), f32)], axis=0)
    vec_all = jnp.pad(vec_all, ((0, 0), (0, D1 - IN_DIM)))
    vh = jnp.concatenate(
        [vec_all[:, q * DQ:(q + 1) * DQ] for q in range(S1)], axis=0)

    x_pad = jnp.concatenate(
        [x[:, 0], jnp.full((NP - N,), NUM_EMB + 1, jnp.int32)])
    emb_idx = jnp.concatenate(
        [x_pad + q * VCAP for q in range(S1)]).reshape(
            S1 * NS, NODE_CHUNKS, ECHUNK)

    src_pad = jnp.concatenate(
        [edge_index[0], jnp.full((EP - E,), N, jnp.int32)])
    dst_pad = jnp.concatenate(
        [edge_index[1], jnp.full((EP - E,), N, jnp.int32)])
    srcw = src_pad.reshape(-1, ECHUNK)
    dstw = dst_pad.reshape(-1, ECHUNK)

    bt = jnp.concatenate(
        [batch, jnp.full((NP - N,), NUM_GRAPHS, jnp.int32)]).reshape(1, NP)

    z2 = _emb_gather(vh, emb_idx)

    zs = []
    zouts = []
    for i in range(NUM_LAYERS):
        (W1, b1, W2, b2), (gamma, beta) = mlp_params[i], bn_params[i]
        first = i == 0
        if first:
            W1 = jnp.pad(W1, ((0, D1 - IN_DIM), (0, 0)))
        a2 = _edge_agg(z2, srcw, dstw, NQ1 if first else 1,
                       S1 if first else S2, 8)
        z2 = _dense(z2, a2, W1, b1.reshape(1, HID), W2,
                    b2.reshape(1, HID), gamma.reshape(1, HID),
                    beta.reshape(1, HID), first)
        zouts.append(z2)
        zs.append(jnp.concatenate([z2[:N], z2[NP:NP + N]], axis=1))
    gs = _pool(zouts[0], zouts[1], zouts[2], bt)
    return (jnp.concatenate(zs, axis=1), gs)


# speedup vs baseline: 2.8790x; 1.0983x over previous
import functools

import jax
import jax.numpy as jnp
from jax import lax
from jax.experimental import pallas as pl
from jax.experimental.pallas import tpu as pltpu
from jax.experimental.pallas import tpu_sc as plsc

N = 10000
E = 160000
NUM_EMB = 11868
IN_DIM = 300
HID = 128
NUM_LAYERS = 3
NUM_GRAPHS = 128

NC = 2
NS = 16
NP = 10240
VCAP = NUM_EMB + 2
D1 = 320
DQ = 64
S1 = D1 // DQ
NQ1 = 3
S2 = HID // DQ
EP = 163840
ECHUNK = 128
EROWS = EP // ECHUNK
ROWS_PER_TILE = NP // NS
NODE_CHUNKS = ROWS_PER_TILE // ECHUNK

_SC_PARAMS = pltpu.CompilerParams(use_tc_tiling_on_sc=False)
_ZV = 16


def _zero_rows(buf, nrow, dq):
    zv = jnp.zeros((_ZV,), jnp.float32)

    def row(i, c):
        for k in range(dq // _ZV):
            buf[i, pl.ds(k * _ZV, _ZV)] = zv
        return c

    lax.fori_loop(0, nrow, row, 0)


def _emb_body(vh_hbm, idx_hbm, out_hbm, idx_v, rows_v, gsem):
    h = lax.axis_index("c")
    s = lax.axis_index("s")
    for ql in range(NQ1):
        q = h * NQ1 + ql

        @pl.when(q < S1)
        def _():
            pltpu.sync_copy(idx_hbm.at[q * NS + s], idx_v)
            out_base = q * NP + s * ROWS_PER_TILE
            cur = pltpu.async_copy(vh_hbm.at[idx_v.at[0]], rows_v.at[0], gsem)
            for j in range(NODE_CHUNKS):
                cur.wait()
                if j + 1 < NODE_CHUNKS:
                    nxt = pltpu.async_copy(
                        vh_hbm.at[idx_v.at[j + 1]], rows_v.at[(j + 1) % 2],
                        gsem)
                pltpu.sync_copy(
                    rows_v.at[j % 2],
                    out_hbm.at[pl.ds(out_base + j * ECHUNK, ECHUNK)])
                if j + 1 < NODE_CHUNKS:
                    cur = nxt


@jax.jit
def _emb_gather(vh, idx):
    mesh = plsc.VectorSubcoreMesh(core_axis_name="c", subcore_axis_name="s")
    f = pl.kernel(
        _emb_body,
        out_type=jax.ShapeDtypeStruct((S1 * NP, DQ), jnp.float32),
        mesh=mesh,
        scratch_types=[
            pltpu.VMEM((NODE_CHUNKS, ECHUNK), jnp.int32),
            pltpu.VMEM((2, ECHUNK, DQ), jnp.float32),
            pltpu.SemaphoreType.DMA,
        ],
        compiler_params=_SC_PARAMS,
    )
    return f(vh, idx)


def _agg_body(nq, ns, dq, nbuf, z_hbm, src_hbm, dst_hbm, out_hbm,
              src_v, dst_v, rows_v, acc, gsem, ssem):
    h = lax.axis_index("c")
    s = lax.axis_index("s")
    nchunk = EROWS // NS
    base = s * nchunk
    pltpu.sync_copy(src_hbm.at[pl.ds(base, nchunk)], src_v)
    pltpu.sync_copy(dst_hbm.at[pl.ds(base, nchunk)], dst_v)

    def add_src(off):
        def adj(i, c):
            for k in range(ECHUNK // _ZV):
                sl = pl.ds(k * _ZV, _ZV)
                src_v[i, sl] = src_v[i, sl] + off
            return c

        lax.fori_loop(0, nchunk, adj, 0)

    nslot = 2 * nbuf

    def gather(j, slot):
        pltpu.async_copy(z_hbm.at[src_v.at[j]], rows_v.at[slot], gsem)

    def scatter(j, slot):
        pltpu.async_copy(rows_v.at[slot], acc.at[dst_v.at[j]], ssem,
                         add=True)

    def wait_gather():
        pltpu.make_async_copy(z_hbm.at[pl.ds(0, ECHUNK)], rows_v.at[0],
                              gsem).wait()

    def wait_scatter():
        pltpu.make_async_copy(rows_v.at[0], acc.at[pl.ds(0, ECHUNK)],
                              ssem).wait()

    def one_pass():
        for j in range(nbuf):
            gather(j, j)

        def steady(j, c):
            wait_gather()
            scatter(j, lax.rem(j, nslot))
            ja = j + nbuf

            @pl.when(j >= nbuf)
            def _():
                wait_scatter()

            gather(ja, lax.rem(ja, nslot))
            return c

        lax.fori_loop(0, nchunk - nbuf, steady, 0)
        for j in range(nchunk - nbuf, nchunk):
            wait_gather()
            scatter(j, j % nslot)
        for _ in range(nslot):
            wait_scatter()

    def zero_acc():
        _zero_rows(rows_v.at[0], ECHUNK, dq)
        for k in range(NODE_CHUNKS):
            pltpu.sync_copy(
                rows_v.at[0],
                acc.at[pl.ds(s * ROWS_PER_TILE + k * ECHUNK, ECHUNK)])

    def dump_acc(q):
        pltpu.sync_copy(
            acc.at[pl.ds(s * ROWS_PER_TILE, ROWS_PER_TILE)],
            out_hbm.at[pl.ds(q * NP + s * ROWS_PER_TILE, ROWS_PER_TILE)])

    for ql in range(nq):
        q = h * nq + ql

        @pl.when(q < ns)
        def _():
            add_src(q * NP if ql == 0 else NP)
            zero_acc()
            plsc.subcore_barrier()
            one_pass()
            plsc.subcore_barrier()
            dump_acc(q)
            if ql + 1 < nq:
                plsc.subcore_barrier()


@functools.partial(jax.jit, static_argnums=(3, 4, 5))
def _edge_agg(z2, srcw, dstw, nq, ns, nbuf):
    mesh = plsc.VectorSubcoreMesh(core_axis_name="c", subcore_axis_name="s")
    nchunk = EROWS // NS
    f = pl.kernel(
        functools.partial(_agg_body, nq, ns, DQ, nbuf),
        out_type=jax.ShapeDtypeStruct((ns * NP, DQ), jnp.float32),
        mesh=mesh,
        scratch_types=[
            pltpu.VMEM((nchunk, ECHUNK), jnp.int32),
            pltpu.VMEM((nchunk, ECHUNK), jnp.int32),
            pltpu.VMEM((nbuf, ECHUNK, DQ), jnp.float32),
            pltpu.VMEM_SHARED((NP, DQ), jnp.float32),
            pltpu.SemaphoreType.DMA,
            pltpu.SemaphoreType.DMA,
        ],
        compiler_params=_SC_PARAMS,
    )
    return f(z2, srcw, dstw)


def _dense_body(nslab, z_ref, a_ref, w1_ref, b1_ref, w2_ref, b2_ref,
                gam_ref, bet_ref, zout_ref, t_acc):
    q = pl.program_id(0)
    part = jnp.dot(z_ref[...] + a_ref[...], w1_ref[...],
                   preferred_element_type=jnp.float32)

    @pl.when(q == 0)
    def _():
        t_acc[...] = b1_ref[...] + part

    @pl.when(q > 0)
    def _():
        t_acc[...] = t_acc[...] + part

    @pl.when(q == nslab - 1)
    def _():
        t = jnp.maximum(t_acc[...], 0.0)
        t = (jnp.dot(t, w2_ref[...], preferred_element_type=jnp.float32)
             + b2_ref[...])
        t = jnp.maximum(t, 0.0)
        mask = lax.broadcasted_iota(jnp.int32, (NP, 1), 0) < N
        tm = jnp.where(mask, t, 0.0)
        mu = jnp.sum(tm, axis=0, keepdims=True) * (1.0 / N)
        d = jnp.where(mask, t - mu, 0.0)
        var = jnp.sum(d * d, axis=0, keepdims=True) * (1.0 / N)
        zz = gam_ref[...] * d * lax.rsqrt(var + 1e-5) + bet_ref[...]
        zz = jnp.where(mask, zz, 0.0)
        zout_ref[pl.ds(0, NP)] = zz[:, :DQ]
        zout_ref[pl.ds(NP, NP)] = zz[:, DQ:]


@functools.partial(jax.jit, static_argnums=(8,))
def _dense(z2, a2, w1, b1, w2, b2, gam, bet, first):
    nslab = S1 if first else S2
    full = lambda *shape: pl.BlockSpec(shape, lambda q: (0,) * len(shape))
    f = pl.pallas_call(
        functools.partial(_dense_body, nslab),
        grid=(nslab,),
        in_specs=[
            pl.BlockSpec((NP, DQ), lambda q: (q, 0)),
            pl.BlockSpec((NP, DQ), lambda q: (q, 0)),
            pl.BlockSpec((DQ, HID), lambda q: (q, 0)),
            full(1, HID), full(HID, HID), full(1, HID),
            full(1, HID), full(1, HID),
        ],
        out_specs=pl.BlockSpec((NC * NP, DQ), lambda q: (0, 0)),
        out_shape=jax.ShapeDtypeStruct((NC * NP, DQ), jnp.float32),
        scratch_shapes=[pltpu.VMEM((NP, HID), jnp.float32)],
    )
    return f(z2, a2, w1, b1, w2, b2, gam, bet)


def _pool_body(z1_ref, z2_ref, z3_ref, bt_ref, g_ref):
    bt = bt_ref[...]
    pt = (lax.broadcasted_iota(jnp.int32, (NUM_GRAPHS, NP), 0)
          == bt).astype(jnp.float32)
    cnt = jnp.sum(pt, axis=1, keepdims=True)
    inv = 1.0 / jnp.maximum(cnt, 1.0)
    for i, zr in enumerate([z1_ref, z2_ref, z3_ref]):
        z = zr[...]
        zz = jnp.concatenate([z[:NP], z[NP:]], axis=1)
        g_ref[:, pl.ds(i * HID, HID)] = inv * lax.dot_general(
            pt, zz, (((1,), (0,)), ((), ())),
            preferred_element_type=jnp.float32)


@jax.jit
def _pool(z1, z2, z3, bt):
    f = pl.pallas_call(
        _pool_body,
        out_shape=jax.ShapeDtypeStruct(
            (NUM_GRAPHS, NUM_LAYERS * HID), jnp.float32),
    )
    return f(z1, z2, z3, bt)


def kernel(x, edge_index, batch, emb, vec_random, mlp_params, bn_params):
    f32 = jnp.float32
    vec_all = jnp.concatenate([emb, vec_random,
                               jnp.zeros((1, IN_DIM), f32)], axis=0)
    vec_all = jnp.pad(vec_all, ((0, 0), (0, D1 - IN_DIM)))
    vh = jnp.concatenate(
        [vec_all[:, q * DQ:(q + 1) * DQ] for q in range(S1)], axis=0)

    x_pad = jnp.concatenate(
        [x[:, 0], jnp.full((NP - N,), NUM_EMB + 1, jnp.int32)])
    emb_idx = jnp.concatenate(
        [x_pad + q * VCAP for q in range(S1)]).reshape(
            S1 * NS, NODE_CHUNKS, ECHUNK)

    src_pad = jnp.concatenate(
        [edge_index[0], jnp.full((EP - E,), N, jnp.int32)])
    dst_pad = jnp.concatenate(
        [edge_index[1], jnp.full((EP - E,), N, jnp.int32)])
    srcw = src_pad.reshape(-1, ECHUNK)
    dstw = dst_pad.reshape(-1, ECHUNK)

    bt = jnp.concatenate(
        [batch, jnp.full((NP - N,), NUM_GRAPHS, jnp.int32)]).reshape(1, NP)

    z2 = _emb_gather(vh, emb_idx)

    zs = []
    zouts = []
    for i in range(NUM_LAYERS):
        (W1, b1, W2, b2), (gamma, beta) = mlp_params[i], bn_params[i]
        first = i == 0
        if first:
            W1 = jnp.pad(W1, ((0, D1 - IN_DIM), (0, 0)))
        a2 = _edge_agg(z2, srcw, dstw, NQ1 if first else 1,
                       S1 if first else S2, 6)
        z2 = _dense(z2, a2, W1, b1.reshape(1, HID), W2,
                    b2.reshape(1, HID), gamma.reshape(1, HID),
                    beta.reshape(1, HID), first)
        zouts.append(z2)
        zs.append(jnp.concatenate([z2[:N], z2[NP:NP + N]], axis=1))
    gs = _pool(zouts[0], zouts[1], zouts[2], bt)
    return (jnp.concatenate(zs, axis=1), gs)


# speedup vs baseline: 2.9226x; 1.0151x over previous
import functools

import jax
import jax.numpy as jnp
from jax import lax
from jax.experimental import pallas as pl
from jax.experimental.pallas import tpu as pltpu
from jax.experimental.pallas import tpu_sc as plsc

N = 10000
E = 160000
NUM_EMB = 11868
IN_DIM = 300
HID = 128
NUM_LAYERS = 3
NUM_GRAPHS = 128

NC = 2
NS = 16
NP = 10240
VCAP = NUM_EMB + 4
D1 = 320
DQ = 32
S1 = D1 // DQ
NQ1 = S1 // NC
DQ2 = DQ
S2 = HID // DQ2
EP = 163840
ECHUNK = 128
EROWS = EP // ECHUNK
ROWS_PER_TILE = NP // NS
NODE_CHUNKS = ROWS_PER_TILE // ECHUNK

_SC_PARAMS = pltpu.CompilerParams(use_tc_tiling_on_sc=False)
_ZV = 16


def _zero_rows(buf, nrow, dq):
    zv = jnp.zeros((_ZV,), jnp.float32)

    def row(i, c):
        for k in range(dq // _ZV):
            buf[i, pl.ds(k * _ZV, _ZV)] = zv
        return c

    lax.fori_loop(0, nrow, row, 0)


def _emb_body(vh_hbm, idx_hbm, out_hbm, idx_v, rows_v, gsem):
    h = lax.axis_index("c")
    s = lax.axis_index("s")
    for ql in range(NQ1):
        q = h * NQ1 + ql

        @pl.when(q < S1)
        def _():
            pltpu.sync_copy(idx_hbm.at[q * NS + s], idx_v)
            out_base = q * NP + s * ROWS_PER_TILE
            cur = pltpu.async_copy(vh_hbm.at[idx_v.at[0]], rows_v.at[0], gsem)
            for j in range(NODE_CHUNKS):
                cur.wait()
                if j + 1 < NODE_CHUNKS:
                    nxt = pltpu.async_copy(
                        vh_hbm.at[idx_v.at[j + 1]], rows_v.at[(j + 1) % 2],
                        gsem)
                pltpu.sync_copy(
                    rows_v.at[j % 2],
                    out_hbm.at[pl.ds(out_base + j * ECHUNK, ECHUNK)])
                if j + 1 < NODE_CHUNKS:
                    cur = nxt


@jax.jit
def _emb_gather(vh, idx):
    mesh = plsc.VectorSubcoreMesh(core_axis_name="c", subcore_axis_name="s")
    f = pl.kernel(
        _emb_body,
        out_type=jax.ShapeDtypeStruct((S1 * NP, DQ), jnp.float32),
        mesh=mesh,
        scratch_types=[
            pltpu.VMEM((NODE_CHUNKS, ECHUNK), jnp.int32),
            pltpu.VMEM((2, ECHUNK, DQ), jnp.float32),
            pltpu.SemaphoreType.DMA,
        ],
        compiler_params=_SC_PARAMS,
    )
    return f(vh, idx)


def _agg_body(nq, ns, dq, nbuf, interleave, z_hbm, src_hbm, dst_hbm, out_hbm,
              src_v, dst_v, rows_v, acc, gsem, ssem):
    h = lax.axis_index("c")
    s = lax.axis_index("s")
    nchunk = EROWS // NS
    base = s * nchunk
    pltpu.sync_copy(src_hbm.at[pl.ds(base, nchunk)], src_v)
    pltpu.sync_copy(dst_hbm.at[pl.ds(base, nchunk)], dst_v)

    def add_src(scale, off):
        def adj(i, c):
            for k in range(ECHUNK // _ZV):
                sl = pl.ds(k * _ZV, _ZV)
                src_v[i, sl] = src_v[i, sl] * scale + off
            return c

        lax.fori_loop(0, nchunk, adj, 0)

    nslot = 2 * nbuf

    def gather(j, slot):
        return pltpu.async_copy(z_hbm.at[src_v.at[j]], rows_v.at[slot], gsem)

    def scatter(j, slot):
        return pltpu.async_copy(rows_v.at[slot], acc.at[dst_v.at[j]], ssem,
                                add=True)

    def one_pass():
        def pair(t, c):
            base = t * 2 * nbuf
            ga = [gather(base + u, u) for u in range(nbuf)]
            for d in ga:
                d.wait()
            sa = [scatter(base + u, u) for u in range(nbuf)]
            gb = [gather(base + nbuf + u, nbuf + u) for u in range(nbuf)]
            for d in gb:
                d.wait()
            for d in sa:
                d.wait()
            sb = [scatter(base + nbuf + u, nbuf + u) for u in range(nbuf)]
            for d in sb:
                d.wait()
            return c

        lax.fori_loop(0, nchunk // (2 * nbuf), pair, 0)

    def zero_acc():
        _zero_rows(rows_v.at[0], ECHUNK, dq)
        for k in range(NODE_CHUNKS):
            pltpu.sync_copy(
                rows_v.at[0],
                acc.at[pl.ds(s * ROWS_PER_TILE + k * ECHUNK, ECHUNK)])

    def dump_acc(q):
        pltpu.sync_copy(
            acc.at[pl.ds(s * ROWS_PER_TILE, ROWS_PER_TILE)],
            out_hbm.at[pl.ds(q * NP + s * ROWS_PER_TILE, ROWS_PER_TILE)])

    for ql in range(nq):
        q = h * nq + ql

        @pl.when(q < ns)
        def _():
            if interleave:
                add_src(ns if ql == 0 else 1, q if ql == 0 else 1)
            else:
                add_src(1, q * NP if ql == 0 else NP)
            zero_acc()
            plsc.subcore_barrier()
            one_pass()
            plsc.subcore_barrier()
            dump_acc(q)
            if ql + 1 < nq:
                plsc.subcore_barrier()


@functools.partial(jax.jit, static_argnums=(3, 4, 5, 6, 7))
def _edge_agg(z2, srcw, dstw, nq, ns, dq, nbuf, interleave):
    mesh = plsc.VectorSubcoreMesh(core_axis_name="c", subcore_axis_name="s")
    nchunk = EROWS // NS
    f = pl.kernel(
        functools.partial(_agg_body, nq, ns, dq, nbuf, interleave),
        out_type=jax.ShapeDtypeStruct((ns * NP, dq), jnp.float32),
        mesh=mesh,
        scratch_types=[
            pltpu.VMEM((nchunk, ECHUNK), jnp.int32),
            pltpu.VMEM((nchunk, ECHUNK), jnp.int32),
            pltpu.VMEM((2 * nbuf, ECHUNK, dq), jnp.float32),
            pltpu.VMEM_SHARED((NP, dq), jnp.float32),
            pltpu.SemaphoreType.DMA,
            pltpu.SemaphoreType.DMA,
        ],
        compiler_params=_SC_PARAMS,
    )
    return f(z2, srcw, dstw)


def _dense_body(nslab, first, z_ref, a_ref, w1_ref, w1f_ref, b1_ref, w2_ref,
                b2_ref, gam_ref, bet_ref, zout_ref, t_acc):
    q = pl.program_id(0)
    if first:
        part = jnp.dot(z_ref[...] + a_ref[...], w1_ref[...],
                       preferred_element_type=jnp.float32)
        init = b1_ref[...] + part
    else:
        part = jnp.dot(a_ref[...], w1_ref[...],
                       preferred_element_type=jnp.float32)
        init = (b1_ref[...] + part
                + jnp.dot(z_ref[...], w1f_ref[...],
                          preferred_element_type=jnp.float32))

    @pl.when(q == 0)
    def _():
        t_acc[...] = init

    @pl.when(q > 0)
    def _():
        t_acc[...] = t_acc[...] + part

    @pl.when(q == nslab - 1)
    def _():
        t = jnp.maximum(t_acc[...], 0.0)
        t = (jnp.dot(t, w2_ref[...], preferred_element_type=jnp.float32)
             + b2_ref[...])
        t = jnp.maximum(t, 0.0)
        mask = lax.broadcasted_iota(jnp.int32, (NP, 1), 0) < N
        tm = jnp.where(mask, t, 0.0)
        mu = jnp.sum(tm, axis=0, keepdims=True) * (1.0 / N)
        d = jnp.where(mask, t - mu, 0.0)
        var = jnp.sum(d * d, axis=0, keepdims=True) * (1.0 / N)
        zz = gam_ref[...] * d * lax.rsqrt(var + 1e-5) + bet_ref[...]
        zout_ref[...] = jnp.where(mask, zz, 0.0)


@functools.partial(jax.jit, static_argnums=(8,))
def _dense(z2, a2, w1, b1, w2, b2, gam, bet, first):
    nslab = S1 if first else S2
    full = lambda *shape: pl.BlockSpec(shape, lambda q: (0,) * len(shape))
    zspec = (pl.BlockSpec((NP, DQ), lambda q: (q, 0)) if first
             else full(NP, HID))
    f = pl.pallas_call(
        functools.partial(_dense_body, nslab, first),
        grid=(nslab,),
        in_specs=[
            zspec,
            pl.BlockSpec((NP, DQ), lambda q: (q, 0)),
            pl.BlockSpec((DQ, HID), lambda q: (q, 0)),
            full(HID, HID),
            full(1, HID), full(HID, HID), full(1, HID),
            full(1, HID), full(1, HID),
        ],
        out_specs=pl.BlockSpec((NP, HID), lambda q: (0, 0)),
        out_shape=jax.ShapeDtypeStruct((NP, HID), jnp.float32),
        scratch_shapes=[pltpu.VMEM((NP, HID), jnp.float32)],
    )
    w1f = w1[:HID] if first else w1
    return f(z2, a2, w1, w1f, b1, w2, b2, gam, bet)


def _prep_edges_body(e_ref, src_ref, dst_ref):
    e = e_ref[...]
    pad = jnp.full((EP // ECHUNK - E // ECHUNK, ECHUNK), N, jnp.int32)
    src_ref[...] = jnp.concatenate([e[0], pad], axis=0)
    dst_ref[...] = jnp.concatenate([e[1], pad], axis=0)


@jax.jit
def _prep_edges(e3):
    f = pl.pallas_call(
        _prep_edges_body,
        out_shape=(
            jax.ShapeDtypeStruct((EROWS, ECHUNK), jnp.int32),
            jax.ShapeDtypeStruct((EROWS, ECHUNK), jnp.int32),
        ),
    )
    return f(e3)


def _prep_pad_body(emb_ref, vr_ref, out_ref):
    out_ref[pl.ds(0, NUM_EMB), pl.ds(0, IN_DIM)] = emb_ref[...]
    out_ref[pl.ds(NUM_EMB, 1), pl.ds(0, IN_DIM)] = vr_ref[...]
    out_ref[pl.ds(NUM_EMB + 1, VCAP - NUM_EMB - 1)] = jnp.zeros(
        (VCAP - NUM_EMB - 1, D1), jnp.float32)
    out_ref[pl.ds(0, NUM_EMB + 1), pl.ds(IN_DIM, D1 - IN_DIM)] = jnp.zeros(
        (NUM_EMB + 1, D1 - IN_DIM), jnp.float32)


@jax.jit
def _prep_emb(emb, vr):
    padded = pl.pallas_call(
        _prep_pad_body,
        out_shape=jax.ShapeDtypeStruct((VCAP, D1), jnp.float32),
    )(emb, vr)
    return padded.reshape(S1 * VCAP, DQ)


def _asm_body(z1_ref, z2_ref, z3_ref, out_ref):
    for i, zr in enumerate([z1_ref, z2_ref, z3_ref]):
        out_ref[:, pl.ds(i * HID, HID)] = zr[pl.ds(0, N)]


@jax.jit
def _asm(z1, z2, z3):
    f = pl.pallas_call(
        _asm_body,
        out_shape=jax.ShapeDtypeStruct((N, NUM_LAYERS * HID), jnp.float32),
    )
    return f(z1, z2, z3)


def _pool_body(z1_ref, z2_ref, z3_ref, bt_ref, g_ref):
    bt = bt_ref[...]
    pt = (lax.broadcasted_iota(jnp.int32, (NUM_GRAPHS, NP), 0)
          == bt).astype(jnp.float32)
    cnt = jnp.sum(pt, axis=1, keepdims=True)
    inv = 1.0 / jnp.maximum(cnt, 1.0)
    for i, zr in enumerate([z1_ref, z2_ref, z3_ref]):
        zz = zr[...]
        g_ref[:, pl.ds(i * HID, HID)] = inv * lax.dot_general(
            pt, zz, (((1,), (0,)), ((), ())),
            preferred_element_type=jnp.float32)


@jax.jit
def _pool(z1, z2, z3, bt):
    f = pl.pallas_call(
        _pool_body,
        out_shape=jax.ShapeDtypeStruct(
            (NUM_GRAPHS, NUM_LAYERS * HID), jnp.float32),
    )
    return f(z1, z2, z3, bt)


def kernel(x, edge_index, batch, emb, vec_random, mlp_params, bn_params):
    vh = _prep_emb(emb, vec_random)
    srcw, dstw = _prep_edges(edge_index.reshape(2, E // ECHUNK, ECHUNK))

    x_pad = jnp.concatenate(
        [x[:, 0], jnp.full((NP - N,), NUM_EMB + 1, jnp.int32)])
    emb_idx = jnp.concatenate(
        [x_pad * S1 + q for q in range(S1)]).reshape(
            S1 * NS, NODE_CHUNKS, ECHUNK)

    bt = jnp.concatenate(
        [batch, jnp.full((NP - N,), NUM_GRAPHS, jnp.int32)]).reshape(1, NP)

    z2 = _emb_gather(vh, emb_idx)

    zouts = []
    for i in range(NUM_LAYERS):
        (W1, b1, W2, b2), (gamma, beta) = mlp_params[i], bn_params[i]
        first = i == 0
        if first:
            W1 = jnp.pad(W1, ((0, D1 - IN_DIM), (0, 0)))
            zin = z2
            a2 = _edge_agg(zin, srcw, dstw, NQ1, S1, DQ, 5, False)
        else:
            zin = z2
            a2 = _edge_agg(z2.reshape(S2 * NP, DQ2), srcw, dstw,
                           S2 // NC, S2, DQ2, 5, True)
        z2 = _dense(zin, a2, W1, b1.reshape(1, HID), W2,
                    b2.reshape(1, HID), gamma.reshape(1, HID),
                    beta.reshape(1, HID), first)
        zouts.append(z2)
    gs = _pool(zouts[0], zouts[1], zouts[2], bt)
    zs = _asm(zouts[0], zouts[1], zouts[2])
    return (zs, gs)


# speedup vs baseline: 2.9844x; 1.0212x over previous
import functools

import jax
import jax.numpy as jnp
from jax import lax
from jax.experimental import pallas as pl
from jax.experimental.pallas import tpu as pltpu
from jax.experimental.pallas import tpu_sc as plsc

N = 10000
E = 160000
NUM_EMB = 11868
IN_DIM = 300
HID = 128
NUM_LAYERS = 3
NUM_GRAPHS = 128

NC = 2
NS = 16
NP = 10240
VCAP = NUM_EMB + 4
D1 = 320
DQ = 32
S1 = D1 // DQ
NQ1 = S1 // NC
DQ2 = DQ
S2 = HID // DQ2
DQP = 64
S2P = HID // DQP
EP = 163840
ECHUNK = 128
EROWS = EP // ECHUNK
ROWS_PER_TILE = NP // NS
NODE_CHUNKS = ROWS_PER_TILE // ECHUNK

_SC_PARAMS = pltpu.CompilerParams(use_tc_tiling_on_sc=False)
_ZV = 16


def _zero_rows(buf, nrow, dq):
    zv = jnp.zeros((_ZV,), jnp.float32)

    def row(i, c):
        for k in range(dq // _ZV):
            buf[i, pl.ds(k * _ZV, _ZV)] = zv
        return c

    lax.fori_loop(0, nrow, row, 0)


def _emb_body(vh_hbm, idx_hbm, out_hbm, idx_v, rows_v, gsem):
    h = lax.axis_index("c")
    s = lax.axis_index("s")
    for ql in range(NQ1):
        q = h * NQ1 + ql

        @pl.when(q < S1)
        def _():
            pltpu.sync_copy(idx_hbm.at[q * NS + s], idx_v)
            out_base = q * NP + s * ROWS_PER_TILE
            cur = pltpu.async_copy(vh_hbm.at[idx_v.at[0]], rows_v.at[0], gsem)
            for j in range(NODE_CHUNKS):
                cur.wait()
                if j + 1 < NODE_CHUNKS:
                    nxt = pltpu.async_copy(
                        vh_hbm.at[idx_v.at[j + 1]], rows_v.at[(j + 1) % 2],
                        gsem)
                pltpu.sync_copy(
                    rows_v.at[j % 2],
                    out_hbm.at[pl.ds(out_base + j * ECHUNK, ECHUNK)])
                if j + 1 < NODE_CHUNKS:
                    cur = nxt


@jax.jit
def _emb_gather(vh, idx):
    mesh = plsc.VectorSubcoreMesh(core_axis_name="c", subcore_axis_name="s")
    f = pl.kernel(
        _emb_body,
        out_type=jax.ShapeDtypeStruct((S1 * NP, DQ), jnp.float32),
        mesh=mesh,
        scratch_types=[
            pltpu.VMEM((NODE_CHUNKS, ECHUNK), jnp.int32),
            pltpu.VMEM((2, ECHUNK, DQ), jnp.float32),
            pltpu.SemaphoreType.DMA,
        ],
        compiler_params=_SC_PARAMS,
    )
    return f(vh, idx)


def _agg_body(nq, ns, dq, nbuf, interleave, z_hbm, src_hbm, dst_hbm, out_hbm,
              src_v, dst_v, rows_v, acc, gsem, ssem):
    h = lax.axis_index("c")
    s = lax.axis_index("s")
    nchunk = EROWS // NS
    base = s * nchunk
    pltpu.sync_copy(src_hbm.at[pl.ds(base, nchunk)], src_v)
    pltpu.sync_copy(dst_hbm.at[pl.ds(base, nchunk)], dst_v)

    def add_src(scale, off):
        def adj(i, c):
            for k in range(ECHUNK // _ZV):
                sl = pl.ds(k * _ZV, _ZV)
                src_v[i, sl] = src_v[i, sl] * scale + off
            return c

        lax.fori_loop(0, nchunk, adj, 0)

    nslot = 2 * nbuf

    def gather(j, slot):
        return pltpu.async_copy(z_hbm.at[src_v.at[j]], rows_v.at[slot], gsem)

    def scatter(j, slot):
        return pltpu.async_copy(rows_v.at[slot], acc.at[dst_v.at[j]], ssem,
                                add=True)

    def one_pass():
        def pair(t, c):
            base = t * 2 * nbuf
            ga = [gather(base + u, u) for u in range(nbuf)]
            for d in ga:
                d.wait()
            sa = [scatter(base + u, u) for u in range(nbuf)]
            gb = [gather(base + nbuf + u, nbuf + u) for u in range(nbuf)]
            for d in gb:
                d.wait()
            for d in sa:
                d.wait()
            sb = [scatter(base + nbuf + u, nbuf + u) for u in range(nbuf)]
            for d in sb:
                d.wait()
            return c

        lax.fori_loop(0, nchunk // (2 * nbuf), pair, 0)

    def zero_acc():
        _zero_rows(rows_v.at[0], ECHUNK, dq)
        for k in range(NODE_CHUNKS):
            pltpu.sync_copy(
                rows_v.at[0],
                acc.at[pl.ds(s * ROWS_PER_TILE + k * ECHUNK, ECHUNK)])

    def dump_acc(q):
        pltpu.sync_copy(
            acc.at[pl.ds(s * ROWS_PER_TILE, ROWS_PER_TILE)],
            out_hbm.at[pl.ds(q * NP + s * ROWS_PER_TILE, ROWS_PER_TILE)])

    for ql in range(nq):
        q = h * nq + ql

        @pl.when(q < ns)
        def _():
            if interleave:
                add_src(ns if ql == 0 else 1, q if ql == 0 else 1)
            else:
                add_src(1, q * NP if ql == 0 else NP)
            zero_acc()
            plsc.subcore_barrier()
            one_pass()
            plsc.subcore_barrier()
            dump_acc(q)
            if ql + 1 < nq:
                plsc.subcore_barrier()


@functools.partial(jax.jit, static_argnums=(3, 4, 5, 6, 7))
def _edge_agg(z2, srcw, dstw, nq, ns, dq, nbuf, interleave):
    mesh = plsc.VectorSubcoreMesh(core_axis_name="c", subcore_axis_name="s")
    nchunk = EROWS // NS
    f = pl.kernel(
        functools.partial(_agg_body, nq, ns, dq, nbuf, interleave),
        out_type=jax.ShapeDtypeStruct((ns * NP, dq), jnp.float32),
        mesh=mesh,
        scratch_types=[
            pltpu.VMEM((nchunk, ECHUNK), jnp.int32),
            pltpu.VMEM((nchunk, ECHUNK), jnp.int32),
            pltpu.VMEM((2 * nbuf, ECHUNK, dq), jnp.float32),
            pltpu.VMEM_SHARED((NP, dq), jnp.float32),
            pltpu.SemaphoreType.DMA,
            pltpu.SemaphoreType.DMA,
        ],
        compiler_params=_SC_PARAMS,
    )
    return f(z2, srcw, dstw)


def _dense_body(nslab, first, z_ref, a_ref, w1_ref, w1f_ref, b1_ref, w2_ref,
                b2_ref, gam_ref, bet_ref, zout_ref, t_acc):
    q = pl.program_id(0)
    if first:
        part = jnp.dot(z_ref[...] + a_ref[...], w1_ref[...],
                       preferred_element_type=jnp.float32)
        init = b1_ref[...] + part
    else:
        part = jnp.dot(a_ref[...], w1_ref[...],
                       preferred_element_type=jnp.float32)
        init = (b1_ref[...] + part
                + jnp.dot(z_ref[...], w1f_ref[...],
                          preferred_element_type=jnp.float32))

    @pl.when(q == 0)
    def _():
        t_acc[...] = init

    @pl.when(q > 0)
    def _():
        t_acc[...] = t_acc[...] + part

    @pl.when(q == nslab - 1)
    def _():
        t = jnp.maximum(t_acc[...], 0.0)
        t = (jnp.dot(t, w2_ref[...], preferred_element_type=jnp.float32)
             + b2_ref[...])
        t = jnp.maximum(t, 0.0)
        mask = lax.broadcasted_iota(jnp.int32, (NP, 1), 0) < N
        tm = jnp.where(mask, t, 0.0)
        mu = jnp.sum(tm, axis=0, keepdims=True) * (1.0 / N)
        d = jnp.where(mask, t - mu, 0.0)
        var = jnp.sum(d * d, axis=0, keepdims=True) * (1.0 / N)
        zz = gam_ref[...] * d * lax.rsqrt(var + 1e-5) + bet_ref[...]
        zout_ref[...] = jnp.where(mask, zz, 0.0)


@functools.partial(jax.jit, static_argnums=(8,))
def _dense(z2, a2, w1, b1, w2, b2, gam, bet, first):
    nslab = S1 if first else S2
    dqa = DQ if first else DQ2
    full = lambda *shape: pl.BlockSpec(shape, lambda q: (0,) * len(shape))
    zspec = (pl.BlockSpec((NP, DQ), lambda q: (q, 0)) if first
             else full(NP, HID))
    f = pl.pallas_call(
        functools.partial(_dense_body, nslab, first),
        grid=(nslab,),
        in_specs=[
            zspec,
            pl.BlockSpec((NP, dqa), lambda q: (q, 0)),
            pl.BlockSpec((dqa, HID), lambda q: (q, 0)),
            full(HID, HID),
            full(1, HID), full(HID, HID), full(1, HID),
            full(1, HID), full(1, HID),
        ],
        out_specs=pl.BlockSpec((NP, HID), lambda q: (0, 0)),
        out_shape=jax.ShapeDtypeStruct((NP, HID), jnp.float32),
        scratch_shapes=[pltpu.VMEM((NP, HID), jnp.float32)],
    )
    w1f = w1[:HID] if first else w1
    return f(z2, a2, w1, w1f, b1, w2, b2, gam, bet)


def _prep_edges_body(e_ref, src_ref, dst_ref):
    e = e_ref[...]
    pad = jnp.full((EP // ECHUNK - E // ECHUNK, ECHUNK), N, jnp.int32)
    src_ref[...] = jnp.concatenate([e[0], pad], axis=0)
    dst_ref[...] = jnp.concatenate([e[1], pad], axis=0)


@jax.jit
def _prep_edges(e3):
    f = pl.pallas_call(
        _prep_edges_body,
        out_shape=(
            jax.ShapeDtypeStruct((EROWS, ECHUNK), jnp.int32),
            jax.ShapeDtypeStruct((EROWS, ECHUNK), jnp.int32),
        ),
    )
    return f(e3)


def _prep_pad_body(emb_ref, vr_ref, out_ref):
    out_ref[pl.ds(0, NUM_EMB), pl.ds(0, IN_DIM)] = emb_ref[...]
    out_ref[pl.ds(NUM_EMB, 1), pl.ds(0, IN_DIM)] = vr_ref[...]
    out_ref[pl.ds(NUM_EMB + 1, VCAP - NUM_EMB - 1)] = jnp.zeros(
        (VCAP - NUM_EMB - 1, D1), jnp.float32)
    out_ref[pl.ds(0, NUM_EMB + 1), pl.ds(IN_DIM, D1 - IN_DIM)] = jnp.zeros(
        (NUM_EMB + 1, D1 - IN_DIM), jnp.float32)


@jax.jit
def _prep_emb(emb, vr):
    padded = pl.pallas_call(
        _prep_pad_body,
        out_shape=jax.ShapeDtypeStruct((VCAP, D1), jnp.float32),
    )(emb, vr)
    return padded.reshape(S1 * VCAP, DQ)


def _prep_idx_body(x_ref, b_ref, idx_ref, bt_ref):
    xp = jnp.concatenate(
        [x_ref[...], jnp.full((1, NP - N), NUM_EMB + 1, jnp.int32)], axis=1)
    for q in range(S1):
        idx_ref[pl.ds(q, 1)] = xp * S1 + q
    bt_ref[...] = jnp.concatenate(
        [b_ref[...], jnp.full((1, NP - N), NUM_GRAPHS, jnp.int32)], axis=1)


@jax.jit
def _prep_idx(xr, br):
    f = pl.pallas_call(
        _prep_idx_body,
        out_shape=(
            jax.ShapeDtypeStruct((S1, NP), jnp.int32),
            jax.ShapeDtypeStruct((1, NP), jnp.int32),
        ),
    )
    return f(xr, br)


def _asm_body(z1_ref, z2_ref, z3_ref, out_ref):
    for i, zr in enumerate([z1_ref, z2_ref, z3_ref]):
        out_ref[:, pl.ds(i * HID, HID)] = zr[pl.ds(0, N)]


@jax.jit
def _asm(z1, z2, z3):
    f = pl.pallas_call(
        _asm_body,
        out_shape=jax.ShapeDtypeStruct((N, NUM_LAYERS * HID), jnp.float32),
    )
    return f(z1, z2, z3)


def _pool_body(z1_ref, z2_ref, z3_ref, bt_ref, g_ref):
    bt = bt_ref[...]
    pt = (lax.broadcasted_iota(jnp.int32, (NUM_GRAPHS, NP), 0)
          == bt).astype(jnp.float32)
    cnt = jnp.sum(pt, axis=1, keepdims=True)
    inv = 1.0 / jnp.maximum(cnt, 1.0)
    for i, zr in enumerate([z1_ref, z2_ref, z3_ref]):
        zz = zr[...]
        g_ref[:, pl.ds(i * HID, HID)] = inv * lax.dot_general(
            pt, zz, (((1,), (0,)), ((), ())),
            preferred_element_type=jnp.float32)


@jax.jit
def _pool(z1, z2, z3, bt):
    f = pl.pallas_call(
        _pool_body,
        out_shape=jax.ShapeDtypeStruct(
            (NUM_GRAPHS, NUM_LAYERS * HID), jnp.float32),
    )
    return f(z1, z2, z3, bt)


def kernel(x, edge_index, batch, emb, vec_random, mlp_params, bn_params):
    vh = _prep_emb(emb, vec_random)
    srcw, dstw = _prep_edges(edge_index.reshape(2, E // ECHUNK, ECHUNK))

    idx2, bt = _prep_idx(x.reshape(1, N), batch.reshape(1, N))
    emb_idx = idx2.reshape(S1 * NS, NODE_CHUNKS, ECHUNK)

    z2 = _emb_gather(vh, emb_idx)

    zouts = []
    for i in range(NUM_LAYERS):
        (W1, b1, W2, b2), (gamma, beta) = mlp_params[i], bn_params[i]
        first = i == 0
        if first:
            W1 = jnp.pad(W1, ((0, D1 - IN_DIM), (0, 0)))
            zin = z2
            a2 = _edge_agg(zin, srcw, dstw, NQ1, S1, DQ, 10, False)
        else:
            zin = z2
            a2 = _edge_agg(z2.reshape(S2 * NP, DQ2), srcw, dstw,
                           S2 // NC, S2, DQ2, 10, True)
        z2 = _dense(zin, a2, W1, b1.reshape(1, HID), W2,
                    b2.reshape(1, HID), gamma.reshape(1, HID),
                    beta.reshape(1, HID), first)
        zouts.append(z2)
    gs = _pool(zouts[0], zouts[1], zouts[2], bt)
    zs = _asm(zouts[0], zouts[1], zouts[2])
    return (zs, gs)
